# baseline (speedup 1.0000x reference)
import functools

import jax
import jax.numpy as jnp
from jax import lax
from jax.experimental import pallas as pl
from jax.experimental.pallas import tpu as pltpu
from jax.experimental.pallas import tpu_sc as plsc

N = 10000
D = 128
E = 320000
NC, NS = 2, 16
NW = NC * NS
CHUNK = 128
NCH = 2560
CPW = NCH // NW
NP = 10240
CUT = 7168
NB = NP - CUT
DUMP = 1024
ACC = CUT + DUMP

_MESH = plsc.VectorSubcoreMesh(core_axis_name="c", subcore_axis_name="s")


def _zero_vmem_2d(ref, rows, cols):
    def body(i, _):
        r = i // (cols // 16)
        k = i % (cols // 16)
        ref[r, pl.ds(k * 16, 16)] = jnp.zeros((16,), jnp.float32)
        return 0
    lax.fori_loop(0, rows * (cols // 16), body, 0)


def _zero_vmem_1d(ref, n):
    def body(i, _):
        ref[pl.ds(i * 16, 16)] = jnp.zeros((16,), jnp.float32)
        return 0
    lax.fori_loop(0, n // 16, body, 0)


NBUF = 2


@functools.partial(
    pl.kernel,
    out_type=[jax.ShapeDtypeStruct((NC, CUT, D), jnp.float32),
              jax.ShapeDtypeStruct((NC, NB, D), jnp.float32)],
    mesh=_MESH,
    scratch_types=[
        pltpu.VMEM((CPW, CHUNK), jnp.int32),
        pltpu.VMEM((NBUF, CHUNK), jnp.int32),
        pltpu.VMEM((NBUF, CHUNK), jnp.int32),
    ] + [pltpu.VMEM((CHUNK, D), jnp.float32)] * NBUF + [
        pltpu.VMEM((8, D), jnp.float32),
        pltpu.VMEM_SHARED((ACC, D), jnp.float32),
    ] + [pltpu.SemaphoreType.DMA] * (2 * NBUF),
)
def _sc_pass(t_h, pidx_h, outa_h, outb_h, pidx_v, gbuf_v, smod_v, *rest):
    rows_b = rest[:NBUF]
    zer_v = rest[NBUF]
    acc_s = rest[NBUF + 1]
    sem_gs = rest[NBUF + 2: NBUF + 2 + NBUF]
    sem_ss = rest[NBUF + 2 + NBUF:]
    c = lax.axis_index("c")
    s = lax.axis_index("s")
    w = s * NC + c

    _zero_vmem_2d(zer_v, 8, D)

    pltpu.sync_copy(pidx_h.at[w], pidx_v)

    lanes = lax.iota(jnp.int32, 16)

    def gather_start(b):
        pltpu.async_copy(t_h.at[gbuf_v.at[b]], rows_b[b], sem_gs[b])

    def gather_wait(b):
        pltpu.make_async_copy(t_h.at[gbuf_v.at[b]], rows_b[b],
                              sem_gs[b]).wait()

    def scatter_start(b):
        pltpu.async_copy(rows_b[b], acc_s.at[smod_v.at[b]], sem_ss[b],
                         add=True)

    def scatter_wait(b):
        pltpu.make_async_copy(rows_b[b], acc_s.at[smod_v.at[b]],
                              sem_ss[b]).wait()

    for phase in range(2):
        live = CUT if phase == 0 else NB
        span = (live + DUMP) // NS

        def zrow(i, _):
            pltpu.sync_copy(zer_v, acc_s.at[pl.ds(s * span + i * 8, 8)])
            return 0
        lax.fori_loop(0, span // 8, zrow, 0)

        plsc.subcore_barrier()

        def unpack(j, b):
            for k in range(CHUNK // 16):
                p = pidx_v[j, pl.ds(k * 16, 16)]
                g = p & (16384 - 1)
                d = p >> 14
                dump = live + ((lanes + j * 7 + k * 16 + s * 61) & (DUMP - 1))
                if phase == 0:
                    dmod = jnp.where(d < CUT, d, dump)
                else:
                    dmod = jnp.where(d >= CUT, d - CUT, dump)
                gbuf_v[b, pl.ds(k * 16, 16)] = g
                smod_v[b, pl.ds(k * 16, 16)] = dmod

        def issue(j, b):
            unpack(j, b)
            gather_start(b)

        def process(b):
            gather_wait(b)
            scatter_start(b)

        def recycle(b):
            scatter_wait(b)

        for b in range(NBUF):
            issue(b, b)

        def round_body(r, _):
            j0 = r * NBUF
            for b in range(NBUF):
                process(b)
            for b in range(NBUF):
                recycle(b)
                issue(j0 + NBUF + b, b)
            return 0

        lax.fori_loop(0, CPW // NBUF - 1, round_body, 0)

        process(0); recycle(0)
        process(1); recycle(1)

        plsc.subcore_barrier()

        wrows = live // NS
        out_h = outa_h if phase == 0 else outb_h
        pltpu.sync_copy(acc_s.at[pl.ds(s * wrows, wrows)],
                        out_h.at[c, pl.ds(s * wrows, wrows)])

        plsc.subcore_barrier()


RPT = NP // NS


@functools.partial(
    pl.kernel,
    out_type=jax.ShapeDtypeStruct((NC, 2, NP), jnp.float32),
    mesh=_MESH,
    scratch_types=[
        pltpu.VMEM((125, 80), jnp.int32),
        pltpu.VMEM((125, 80), jnp.int32),
        pltpu.VMEM((80,), jnp.float32),
        pltpu.VMEM((RPT,), jnp.float32),
        pltpu.VMEM_SHARED((NP,), jnp.float32),
        pltpu.VMEM_SHARED((NP,), jnp.float32),
    ],
)
def _sc_degrees(nidx_h, eidx_h, out_h, nidx_v, eidx_v, ones_v, zer_v,
                dv_s, de_s):
    c = lax.axis_index("c")
    s = lax.axis_index("s")
    w = s * NC + c

    for i in range(80 // 16):
        ones_v[pl.ds(i * 16, 16)] = jnp.ones((16,), jnp.float32)
    _zero_vmem_1d(zer_v, RPT)
    pltpu.sync_copy(zer_v, dv_s.at[pl.ds(s * RPT, RPT)])
    pltpu.sync_copy(zer_v, de_s.at[pl.ds(s * RPT, RPT)])

    pltpu.sync_copy(nidx_h.at[w], nidx_v)
    pltpu.sync_copy(eidx_h.at[w], eidx_v)

    plsc.subcore_barrier()

    def chunk(j, _):
        pltpu.sync_copy(ones_v, dv_s.at[nidx_v.at[j]], add=True)
        pltpu.sync_copy(ones_v, de_s.at[eidx_v.at[j]], add=True)
        return 0

    lax.fori_loop(0, 125, chunk, 0)

    plsc.subcore_barrier()

    pltpu.sync_copy(dv_s.at[pl.ds(s * RPT, RPT)],
                    out_h.at[c, 0, pl.ds(s * RPT, RPT)])
    pltpu.sync_copy(de_s.at[pl.ds(s * RPT, RPT)],
                    out_h.at[c, 1, pl.ds(s * RPT, RPT)])


_RB = 1024
_NA = CUT // _RB


def _combine1_body(a0, a1, b0, b1, s1, o1):
    i = pl.program_id(0)

    @pl.when(i < _NA)
    def _():
        o1[...] = (a0[...] + a1[...]) * s1[...]

    @pl.when(i >= _NA)
    def _():
        o1[...] = (b0[...] + b1[...]) * s1[...]


def _combine2_body(a0, a1, b0, b1, s1, s2, o1, o2):
    i = pl.program_id(0)

    @pl.when(i < _NA)
    def _():
        t = a0[...] + a1[...]
        o1[...] = t * s1[...]
        o2[...] = t * s2[...]

    @pl.when(i >= _NA)
    def _():
        t = b0[...] + b1[...]
        o1[...] = t * s1[...]
        o2[...] = t * s2[...]


def _row_spec(cols=D):
    return pl.BlockSpec((_RB, cols), lambda i: (i, 0))


def _a_spec():
    return pl.BlockSpec((_RB, D), lambda i: (jnp.minimum(i, _NA - 1), 0))


def _b_spec():
    return pl.BlockSpec((_RB, D),
                        lambda i: (jnp.clip(i - _NA, 0, NB // _RB - 1), 0))


def _col_spec():
    return pl.BlockSpec((_RB, 1), lambda i: (i, 0))


_TBL = jax.ShapeDtypeStruct((NP, D), jnp.float32)


def _combine1(pa, pb, s1):
    return pl.pallas_call(
        _combine1_body,
        grid=(NP // _RB,),
        in_specs=[_a_spec(), _a_spec(), _b_spec(), _b_spec(), _col_spec()],
        out_specs=_row_spec(),
        out_shape=_TBL,
    )(pa[0], pa[1], pb[0], pb[1], s1)


def _combine2(pa, pb, s1, s2):
    return pl.pallas_call(
        _combine2_body,
        grid=(NP // _RB,),
        in_specs=[_a_spec(), _a_spec(), _b_spec(), _b_spec(),
                  _col_spec(), _col_spec()],
        out_specs=[_row_spec(), _row_spec()],
        out_shape=[_TBL] * 2,
    )(pa[0], pa[1], pb[0], pb[1], s1, s2)


def _invs_body(dp, inv_o, inv2_o):
    deg_v = dp[0] + dp[2]
    deg_e = dp[1] + dp[3]
    iv = jnp.where(deg_v > 0, 1.0 / deg_v, 0.0)
    ie = jnp.where(deg_e > 0, 1.0 / deg_e, 0.0)
    inv_o[0], inv_o[1] = iv, ie
    inv2_o[0], inv2_o[1] = iv * iv, ie * ie


def _invs(degp):
    sp = pl.BlockSpec((2, NP // 128, 128), lambda: (0, 0, 0))
    return pl.pallas_call(
        _invs_body,
        in_specs=[pl.BlockSpec((4, NP // 128, 128), lambda: (0, 0, 0))],
        out_specs=[sp, sp],
        out_shape=[jax.ShapeDtypeStruct((2, NP // 128, 128), jnp.float32)] * 2,
    )(degp)


def _assemble_body(l0, l1, l2, l3, l4, l5, cf, out):
    refs = (l0, l1, l2, l3, l4, l5)
    for s in range(6):
        a, b = s, min(s + 1, 5)
        out[:, s * D:(s + 1) * D] = (refs[a][...] * cf[s, 0]
                                     + refs[b][...] * cf[s, 1])


def _assemble(levels, coef):
    return pl.pallas_call(
        _assemble_body,
        grid=(NP // _RB,),
        in_specs=[_row_spec()] * 6 + [pl.BlockSpec(memory_space=pltpu.SMEM)],
        out_specs=_row_spec(6 * D),
        out_shape=jax.ShapeDtypeStruct((NP, 6 * D), jnp.float32),
    )(*levels, coef)


def kernel(x, hyperedge_index, hyperedge_attr, num_edges,
           wavelet_constructor):
    nidx = hyperedge_index[0].reshape(E // CHUNK, CHUNK)
    eidx = hyperedge_index[1].reshape(E // CHUNK, CHUNK)
    npad = NCH - E // CHUNK
    prow = jnp.arange(npad)[:, None] * CHUNK + jnp.arange(CHUNK)[None, :]
    padc = (prow % N) | ((NP + (prow * 13) % DUMP) << 14)
    pk1 = jnp.concatenate([nidx | (eidx << 14),
                           padc.astype(jnp.int32)]).reshape(NW, CPW, CHUNK)
    pk2 = jnp.concatenate([eidx | (nidx << 14),
                           padc.astype(jnp.int32)]).reshape(NW, CPW, CHUNK)

    degp = _sc_degrees(hyperedge_index[0].reshape(NW, 125, 80),
                       hyperedge_index[1].reshape(NW, 125, 80))
    inv, inv2 = _invs(degp.reshape(NC * 2, NP // 128, 128))
    dv = inv.reshape(2, NP)[0].reshape(NP, 1)
    de = inv.reshape(2, NP)[1].reshape(NP, 1)
    dv2 = inv2.reshape(2, NP)[0].reshape(NP, 1)
    de2 = inv2.reshape(2, NP)[1].reshape(NP, 1)

    padr = jnp.zeros((NP - N, D), jnp.float32)
    x_p = jnp.concatenate([x, padr])
    attr_p = jnp.concatenate([hyperedge_attr, padr])
    z = jnp.zeros((NP, D), jnp.float32)

    yhat = _combine1((x_p[:CUT], z[:CUT]), (x_p[CUT:], z[CUT:]), dv)

    ck1 = (0, 1, 3, 7, 15)
    node_ck = [x_p]
    edge_ck = [attr_p]
    for l in range(16):
        pa, pb = _sc_pass(yhat, pk1)
        if l in ck1:
            ehat, oe = _combine2(pa, pb, de2, de)
            edge_ck.append(oe)
        else:
            ehat = _combine1(pa, pb, de2)
        pa, pb = _sc_pass(ehat, pk2)
        if l == 15:
            node_ck.append(_combine1(pa, pb, dv))
        elif l in ck1:
            yhat, xl = _combine2(pa, pb, dv2, dv)
            node_ck.append(xl)
        else:
            yhat = _combine1(pa, pb, dv2)

    W = wavelet_constructor
    c0 = W[jnp.arange(6), jnp.array([0, 1, 2, 4, 8, 16])]
    c1 = jnp.concatenate([W[jnp.arange(5), jnp.array([1, 2, 4, 8, 16])],
                          jnp.zeros((1,), W.dtype)])
    coef = jnp.stack([c0, c1], axis=1)

    node_emb = _assemble(node_ck, coef)[:N]
    edge_emb = _assemble(edge_ck, coef)[:N]
    return (node_emb, edge_emb)


# speedup vs baseline: 1.2775x; 1.2775x over previous
import functools

import jax
import jax.numpy as jnp
from jax import lax
from jax.experimental import pallas as pl
from jax.experimental.pallas import tpu as pltpu
from jax.experimental.pallas import tpu_sc as plsc

N = 10000
D = 128
E = 320000
NC, NS = 2, 16
NW = NC * NS
CHUNK = 48
NCH = 6720
CPW = NCH // NW
NP = 10240
CUT = 7168
NB = NP - CUT
DUMP = 1024
ACC = CUT + DUMP

_MESH = plsc.VectorSubcoreMesh(core_axis_name="c", subcore_axis_name="s")


def _zero_vmem_2d(ref, rows, cols):
    def body(i, _):
        r = i // (cols // 16)
        k = i % (cols // 16)
        ref[r, pl.ds(k * 16, 16)] = jnp.zeros((16,), jnp.float32)
        return 0
    lax.fori_loop(0, rows * (cols // 16), body, 0)


def _zero_vmem_1d(ref, n):
    def body(i, _):
        ref[pl.ds(i * 16, 16)] = jnp.zeros((16,), jnp.float32)
        return 0
    lax.fori_loop(0, n // 16, body, 0)


NBUF = 5


@functools.partial(
    pl.kernel,
    out_type=[jax.ShapeDtypeStruct((NC, CUT, D), jnp.float32),
              jax.ShapeDtypeStruct((NC, NB, D), jnp.float32)],
    mesh=_MESH,
    scratch_types=[
        pltpu.VMEM((CPW, CHUNK), jnp.int32),
        pltpu.VMEM((NBUF, CHUNK), jnp.int32),
        pltpu.VMEM((NBUF, CHUNK), jnp.int32),
    ] + [pltpu.VMEM((CHUNK, D), jnp.float32)] * NBUF + [
        pltpu.VMEM((8, D), jnp.float32),
        pltpu.VMEM_SHARED((ACC, D), jnp.float32),
    ] + [pltpu.SemaphoreType.DMA] * (2 * NBUF),
)
def _sc_pass(t_h, pidx_h, outa_h, outb_h, pidx_v, gbuf_v, smod_v, *rest):
    rows_b = rest[:NBUF]
    zer_v = rest[NBUF]
    acc_s = rest[NBUF + 1]
    sem_gs = rest[NBUF + 2: NBUF + 2 + NBUF]
    sem_ss = rest[NBUF + 2 + NBUF:]
    c = lax.axis_index("c")
    s = lax.axis_index("s")
    w = s * NC + c

    _zero_vmem_2d(zer_v, 8, D)

    pltpu.sync_copy(pidx_h.at[w], pidx_v)

    lanes = lax.iota(jnp.int32, 16)

    def gather_start(b):
        pltpu.async_copy(t_h.at[gbuf_v.at[b]], rows_b[b], sem_gs[b])

    def gather_wait(b):
        pltpu.make_async_copy(t_h.at[gbuf_v.at[b]], rows_b[b],
                              sem_gs[b]).wait()

    def scatter_start(b):
        pltpu.async_copy(rows_b[b], acc_s.at[smod_v.at[b]], sem_ss[b],
                         add=True)

    def scatter_wait(b):
        pltpu.make_async_copy(rows_b[b], acc_s.at[smod_v.at[b]],
                              sem_ss[b]).wait()

    for phase in range(2):
        live = CUT if phase == 0 else NB
        span = (live + DUMP) // NS

        def zrow(i, _):
            pltpu.sync_copy(zer_v, acc_s.at[pl.ds(s * span + i * 8, 8)])
            return 0
        lax.fori_loop(0, span // 8, zrow, 0)

        plsc.subcore_barrier()

        def unpack(j, b):
            for k in range(CHUNK // 16):
                p = pidx_v[j, pl.ds(k * 16, 16)]
                g = p & (16384 - 1)
                d = p >> 14
                dump = live + ((lanes + j * 7 + k * 16 + s * 61) & (DUMP - 1))
                if phase == 0:
                    dmod = jnp.where(d < CUT, d, dump)
                else:
                    dmod = jnp.where(d >= CUT, d - CUT, dump)
                gbuf_v[b, pl.ds(k * 16, 16)] = g
                smod_v[b, pl.ds(k * 16, 16)] = dmod

        def issue(j, b):
            unpack(j, b)
            gather_start(b)

        def process(b):
            gather_wait(b)
            scatter_start(b)

        def recycle(b):
            scatter_wait(b)

        for b in range(NBUF):
            issue(b, b)

        def round_body(r, _):
            j0 = r * NBUF
            for b in range(NBUF):
                process(b)
            for b in range(NBUF):
                recycle(b)
                issue(j0 + NBUF + b, b)
            return 0

        lax.fori_loop(0, CPW // NBUF - 1, round_body, 0)

        for b in range(NBUF):
            process(b)
            recycle(b)

        plsc.subcore_barrier()

        wrows = live // NS
        out_h = outa_h if phase == 0 else outb_h
        pltpu.sync_copy(acc_s.at[pl.ds(s * wrows, wrows)],
                        out_h.at[c, pl.ds(s * wrows, wrows)])

        plsc.subcore_barrier()


RPT = NP // NS


@functools.partial(
    pl.kernel,
    out_type=jax.ShapeDtypeStruct((NC, 2, NP), jnp.float32),
    mesh=_MESH,
    scratch_types=[
        pltpu.VMEM((125, 80), jnp.int32),
        pltpu.VMEM((125, 80), jnp.int32),
        pltpu.VMEM((80,), jnp.float32),
        pltpu.VMEM((RPT,), jnp.float32),
        pltpu.VMEM_SHARED((NP,), jnp.float32),
        pltpu.VMEM_SHARED((NP,), jnp.float32),
    ],
)
def _sc_degrees(nidx_h, eidx_h, out_h, nidx_v, eidx_v, ones_v, zer_v,
                dv_s, de_s):
    c = lax.axis_index("c")
    s = lax.axis_index("s")
    w = s * NC + c

    for i in range(80 // 16):
        ones_v[pl.ds(i * 16, 16)] = jnp.ones((16,), jnp.float32)
    _zero_vmem_1d(zer_v, RPT)
    pltpu.sync_copy(zer_v, dv_s.at[pl.ds(s * RPT, RPT)])
    pltpu.sync_copy(zer_v, de_s.at[pl.ds(s * RPT, RPT)])

    pltpu.sync_copy(nidx_h.at[w], nidx_v)
    pltpu.sync_copy(eidx_h.at[w], eidx_v)

    plsc.subcore_barrier()

    def chunk(j, _):
        pltpu.sync_copy(ones_v, dv_s.at[nidx_v.at[j]], add=True)
        pltpu.sync_copy(ones_v, de_s.at[eidx_v.at[j]], add=True)
        return 0

    lax.fori_loop(0, 125, chunk, 0)

    plsc.subcore_barrier()

    pltpu.sync_copy(dv_s.at[pl.ds(s * RPT, RPT)],
                    out_h.at[c, 0, pl.ds(s * RPT, RPT)])
    pltpu.sync_copy(de_s.at[pl.ds(s * RPT, RPT)],
                    out_h.at[c, 1, pl.ds(s * RPT, RPT)])


_RB = 1024
_NA = CUT // _RB


def _combine1_body(a0, a1, b0, b1, s1, o1):
    i = pl.program_id(0)

    @pl.when(i < _NA)
    def _():
        o1[...] = (a0[...] + a1[...]) * s1[...]

    @pl.when(i >= _NA)
    def _():
        o1[...] = (b0[...] + b1[...]) * s1[...]


def _combine2_body(a0, a1, b0, b1, s1, s2, o1, o2):
    i = pl.program_id(0)

    @pl.when(i < _NA)
    def _():
        t = a0[...] + a1[...]
        o1[...] = t * s1[...]
        o2[...] = t * s2[...]

    @pl.when(i >= _NA)
    def _():
        t = b0[...] + b1[...]
        o1[...] = t * s1[...]
        o2[...] = t * s2[...]


def _row_spec(cols=D):
    return pl.BlockSpec((_RB, cols), lambda i: (i, 0))


def _a_spec():
    return pl.BlockSpec((_RB, D), lambda i: (jnp.minimum(i, _NA - 1), 0))


def _b_spec():
    return pl.BlockSpec((_RB, D),
                        lambda i: (jnp.clip(i - _NA, 0, NB // _RB - 1), 0))


def _col_spec():
    return pl.BlockSpec((_RB, 1), lambda i: (i, 0))


_TBL = jax.ShapeDtypeStruct((NP, D), jnp.float32)


def _combine1(pa, pb, s1):
    return pl.pallas_call(
        _combine1_body,
        grid=(NP // _RB,),
        in_specs=[_a_spec(), _a_spec(), _b_spec(), _b_spec(), _col_spec()],
        out_specs=_row_spec(),
        out_shape=_TBL,
    )(pa[0], pa[1], pb[0], pb[1], s1)


def _combine2(pa, pb, s1, s2):
    return pl.pallas_call(
        _combine2_body,
        grid=(NP // _RB,),
        in_specs=[_a_spec(), _a_spec(), _b_spec(), _b_spec(),
                  _col_spec(), _col_spec()],
        out_specs=[_row_spec(), _row_spec()],
        out_shape=[_TBL] * 2,
    )(pa[0], pa[1], pb[0], pb[1], s1, s2)


def _invs_body(dp, inv_o, inv2_o):
    deg_v = dp[0] + dp[2]
    deg_e = dp[1] + dp[3]
    iv = jnp.where(deg_v > 0, 1.0 / deg_v, 0.0)
    ie = jnp.where(deg_e > 0, 1.0 / deg_e, 0.0)
    inv_o[0], inv_o[1] = iv, ie
    inv2_o[0], inv2_o[1] = iv * iv, ie * ie


def _invs(degp):
    sp = pl.BlockSpec((2, NP // 128, 128), lambda: (0, 0, 0))
    return pl.pallas_call(
        _invs_body,
        in_specs=[pl.BlockSpec((4, NP // 128, 128), lambda: (0, 0, 0))],
        out_specs=[sp, sp],
        out_shape=[jax.ShapeDtypeStruct((2, NP // 128, 128), jnp.float32)] * 2,
    )(degp)


def _assemble_body(l0, l1, l2, l3, l4, l5, cf, out):
    refs = (l0, l1, l2, l3, l4, l5)
    for s in range(6):
        a, b = s, min(s + 1, 5)
        out[:, s * D:(s + 1) * D] = (refs[a][...] * cf[s, 0]
                                     + refs[b][...] * cf[s, 1])


def _assemble(levels, coef):
    return pl.pallas_call(
        _assemble_body,
        grid=(NP // _RB,),
        in_specs=[_row_spec()] * 6 + [pl.BlockSpec(memory_space=pltpu.SMEM)],
        out_specs=_row_spec(6 * D),
        out_shape=jax.ShapeDtypeStruct((NP, 6 * D), jnp.float32),
    )(*levels, coef)


def kernel(x, hyperedge_index, hyperedge_attr, num_edges,
           wavelet_constructor):
    nidx = hyperedge_index[0]
    eidx = hyperedge_index[1]
    npad = NCH * CHUNK - E
    pent = jnp.arange(npad)
    padc = (pent % N) | ((NP + (pent * 13) % DUMP) << 14)
    pk1 = jnp.concatenate([nidx | (eidx << 14),
                           padc.astype(jnp.int32)]).reshape(NW, CPW, CHUNK)
    pk2 = jnp.concatenate([eidx | (nidx << 14),
                           padc.astype(jnp.int32)]).reshape(NW, CPW, CHUNK)

    degp = _sc_degrees(hyperedge_index[0].reshape(NW, 125, 80),
                       hyperedge_index[1].reshape(NW, 125, 80))
    inv, inv2 = _invs(degp.reshape(NC * 2, NP // 128, 128))
    dv = inv.reshape(2, NP)[0].reshape(NP, 1)
    de = inv.reshape(2, NP)[1].reshape(NP, 1)
    dv2 = inv2.reshape(2, NP)[0].reshape(NP, 1)
    de2 = inv2.reshape(2, NP)[1].reshape(NP, 1)

    padr = jnp.zeros((NP - N, D), jnp.float32)
    x_p = jnp.concatenate([x, padr])
    attr_p = jnp.concatenate([hyperedge_attr, padr])
    z = jnp.zeros((NP, D), jnp.float32)

    yhat = _combine1((x_p[:CUT], z[:CUT]), (x_p[CUT:], z[CUT:]), dv)

    ck1 = (0, 1, 3, 7, 15)
    node_ck = [x_p]
    edge_ck = [attr_p]
    for l in range(16):
        pa, pb = _sc_pass(yhat, pk1)
        if l in ck1:
            ehat, oe = _combine2(pa, pb, de2, de)
            edge_ck.append(oe)
        else:
            ehat = _combine1(pa, pb, de2)
        pa, pb = _sc_pass(ehat, pk2)
        if l == 15:
            node_ck.append(_combine1(pa, pb, dv))
        elif l in ck1:
            yhat, xl = _combine2(pa, pb, dv2, dv)
            node_ck.append(xl)
        else:
            yhat = _combine1(pa, pb, dv2)

    W = wavelet_constructor
    c0 = W[jnp.arange(6), jnp.array([0, 1, 2, 4, 8, 16])]
    c1 = jnp.concatenate([W[jnp.arange(5), jnp.array([1, 2, 4, 8, 16])],
                          jnp.zeros((1,), W.dtype)])
    coef = jnp.stack([c0, c1], axis=1)

    node_emb = _assemble(node_ck, coef)[:N]
    edge_emb = _assemble(edge_ck, coef)[:N]
    return (node_emb, edge_emb)
